# Initial kernel scaffold; baseline (speedup 1.0000x reference)
import functools

import jax
import jax.numpy as jnp
from jax import lax
from jax.experimental import pallas as pl
from jax.experimental.pallas import tpu as pltpu
from jax.experimental.pallas import tpu_sc as plsc

_N = 10000
_E = 320000
_D = 128
_DP = 16
_NTILES = 32
_EPT = _E // _NTILES
_CH = 80
_NCH = _EPT // _CH
_RPT = _N // 16
_ZR = 125


def _spmm(d, feats, src, dst, w):
    mesh = plsc.VectorSubcoreMesh(core_axis_name="c", subcore_axis_name="s")
    nseg = d // 16

    @functools.partial(
        pl.kernel,
        out_type=jax.ShapeDtypeStruct((2 * _N, d), jnp.float32),
        mesh=mesh,
        scratch_types=[
            pltpu.VMEM((_CH,), jnp.int32),
            pltpu.VMEM((_CH,), jnp.int32),
            pltpu.VMEM((_CH,), jnp.float32),
            pltpu.VMEM((_CH, d), jnp.float32),
            pltpu.VMEM((_ZR, d), jnp.float32),
            pltpu.VMEM_SHARED((_N, d), jnp.float32),
            pltpu.SemaphoreType.DMA,
        ],
    )
    def k(f_hbm, src_hbm, dst_hbm, w_hbm, out_hbm, srcv, dstv, wv, rows,
          zbuf, acc, gsem):
        cid = lax.axis_index("c")
        sid = lax.axis_index("s")
        wid = cid * 16 + sid
        zero16 = jnp.zeros((16,), jnp.float32)

        def zrow(i, carry):
            for j in range(nseg):
                zbuf[i, pl.ds(j * 16, 16)] = zero16
            return carry

        lax.fori_loop(0, _ZR, zrow, 0)
        for j in range(_RPT // _ZR):
            pltpu.sync_copy(zbuf, acc.at[pl.ds(sid * _RPT + j * _ZR, _ZR)])
        plsc.subcore_barrier()

        def chunk(g, carry):
            base = wid * _EPT + g * _CH
            pltpu.sync_copy(src_hbm.at[pl.ds(base, _CH)], srcv)
            pltpu.sync_copy(dst_hbm.at[pl.ds(base, _CH)], dstv)
            pltpu.sync_copy(w_hbm.at[pl.ds(base, _CH)], wv)
            pltpu.async_copy(f_hbm.at[srcv], rows, gsem).wait()

            def scale(e, c2):
                ws = plsc.load_gather(wv, [jnp.full((16,), e, jnp.int32)])
                for j in range(nseg):
                    rows[e, pl.ds(j * 16, 16)] = rows[e, pl.ds(j * 16, 16)] * ws
                return c2

            lax.fori_loop(0, _CH, scale, 0)
            pltpu.sync_copy(rows, acc.at[dstv], add=True)
            return carry

        lax.fori_loop(0, _NCH, chunk, 0)
        plsc.subcore_barrier()
        for j in range(_RPT // _ZR):
            r = sid * _RPT + j * _ZR
            pltpu.sync_copy(acc.at[pl.ds(r, _ZR)],
                            out_hbm.at[pl.ds(cid * _N + r, _ZR)])

    return k(feats, src, dst, w)


def _mm_body(x_ref, w_ref, o_ref):
    o_ref[...] = jnp.dot(x_ref[...], w_ref[...],
                         preferred_element_type=jnp.float32)


def _tc_mm(x, w):
    m, kdim = x.shape
    n = w.shape[1]
    bm = 1000
    return pl.pallas_call(
        _mm_body,
        grid=(m // bm,),
        in_specs=[pl.BlockSpec((bm, kdim), lambda i: (i, 0)),
                  pl.BlockSpec((kdim, n), lambda i: (0, 0))],
        out_specs=pl.BlockSpec((bm, n), lambda i: (i, 0)),
        out_shape=jax.ShapeDtypeStruct((m, n), jnp.float32),
    )(x, w)


def _mid_body(a_ref, b_ref, m_ref, w_ref, o_ref):
    s = a_ref[0] + a_ref[1] + b_ref[...]
    h = jnp.maximum(s, 0.0) * m_ref[...]
    o_ref[...] = jnp.dot(h, w_ref[...], preferred_element_type=jnp.float32)


def _mid(agg, b1, mask, w2p):
    bm = 1000
    return pl.pallas_call(
        _mid_body,
        grid=(_N // bm,),
        in_specs=[pl.BlockSpec((2, bm, _D), lambda i: (0, i, 0)),
                  pl.BlockSpec((1, _D), lambda i: (0, 0)),
                  pl.BlockSpec((bm, 1), lambda i: (i, 0)),
                  pl.BlockSpec((_D, _DP), lambda i: (0, 0))],
        out_specs=pl.BlockSpec((bm, _DP), lambda i: (i, 0)),
        out_shape=jax.ShapeDtypeStruct((_N, _DP), jnp.float32),
    )(agg, b1, mask, w2p)


def _out_body(a_ref, b_ref, m_ref, o_ref):
    s = a_ref[0] + a_ref[1] + b_ref[...]
    o_ref[...] = jax.nn.sigmoid(s[:, :2]) * m_ref[...]


def _out(agg, b2p, mask):
    bm = 1000
    return pl.pallas_call(
        _out_body,
        grid=(_N // bm,),
        in_specs=[pl.BlockSpec((2, bm, _DP), lambda i: (0, i, 0)),
                  pl.BlockSpec((1, _DP), lambda i: (0, 0)),
                  pl.BlockSpec((bm, 1), lambda i: (i, 0))],
        out_specs=pl.BlockSpec((bm, 2), lambda i: (i, 0)),
        out_shape=jax.ShapeDtypeStruct((_N, 2), jnp.float32),
    )(agg, b2p, mask)


def kernel(x, edge_index, edge_weight, mask, W1, b1, W2, b2):
    src = edge_index[0].astype(jnp.int32)
    dst = edge_index[1].astype(jnp.int32)
    w = edge_weight.astype(jnp.float32)

    h = _tc_mm(x, W1)
    agg1 = _spmm(_D, h, src, dst, w).reshape(2, _N, _D)

    w2p = jnp.zeros((_D, _DP), jnp.float32).at[:, :2].set(W2)
    p = _mid(agg1, b1.reshape(1, _D), mask, w2p)

    agg2 = _spmm(_DP, p, src, dst, w).reshape(2, _N, _DP)
    b2p = jnp.zeros((1, _DP), jnp.float32).at[0, :2].set(b2)
    return _out(agg2, b2p, mask)


# speedup vs baseline: 2.8252x; 2.8252x over previous
import functools

import jax
import jax.numpy as jnp
from jax import lax
from jax.experimental import pallas as pl
from jax.experimental.pallas import tpu as pltpu
from jax.experimental.pallas import tpu_sc as plsc

_N = 10000
_E = 320000
_D = 128
_DP = 16
_DH = 64
_CH = 80
_RCH = 400
_NRCH = _N // _RCH


def _spmm(d, f_tables, src, dst, w, edge_split):
    mesh = plsc.VectorSubcoreMesh(core_axis_name="c", subcore_axis_name="s")
    nseg = d // 16
    ept = _E // 32 if edge_split else _E // 16
    nch = ept // _CH

    @functools.partial(
        pl.kernel,
        out_type=jax.ShapeDtypeStruct((2 * _N, d), jnp.float32),
        mesh=mesh,
        scratch_types=[
            pltpu.VMEM((_CH,), jnp.int32),
            pltpu.VMEM((_CH,), jnp.int32),
            pltpu.VMEM((_CH,), jnp.float32),
            pltpu.VMEM((_CH, d), jnp.float32),
            pltpu.VMEM((_RCH, d), jnp.float32),
            pltpu.VMEM_SHARED((_N, d), jnp.float32),
            pltpu.SemaphoreType.DMA,
        ],
        compiler_params=pltpu.CompilerParams(use_tc_tiling_on_sc=False),
    )
    def k(*refs):
        f_hbms = refs[:len(f_tables)]
        (src_hbm, dst_hbm, w_hbm, out_hbm, srcv, dstv, wv, rows,
         zbuf, acc, gsem) = refs[len(f_tables):]
        cid = lax.axis_index("c")
        sid = lax.axis_index("s")
        zero16 = jnp.zeros((16,), jnp.float32)

        def zrow(i, carry):
            for j in range(nseg):
                zbuf[i, pl.ds(j * 16, 16)] = zero16
            return carry

        lax.fori_loop(0, _RCH, zrow, 0)
        for j in range(2):
            c = sid + j * 16

            @pl.when(c < _NRCH)
            def _():
                pltpu.sync_copy(zbuf, acc.at[pl.ds(c * _RCH, _RCH)])

        plsc.subcore_barrier()

        def chunk(g, carry):
            if edge_split:
                base = (cid * 16 + sid) * ept + g * _CH
            else:
                base = sid * ept + g * _CH
            pltpu.sync_copy(src_hbm.at[pl.ds(base, _CH)], srcv)
            pltpu.sync_copy(dst_hbm.at[pl.ds(base, _CH)], dstv)
            pltpu.sync_copy(w_hbm.at[pl.ds(base, _CH)], wv)
            if len(f_tables) == 1:
                pltpu.async_copy(f_hbms[0].at[srcv], rows, gsem).wait()
            else:
                for c in range(2):
                    @pl.when(cid == c)
                    def _():
                        pltpu.async_copy(f_hbms[c].at[srcv], rows,
                                         gsem).wait()

            def scale(g16, c2):
                wvec = wv[pl.ds(g16 * 16, 16)]
                for l in range(16):
                    e = g16 * 16 + l
                    ws = jnp.full((16,), wvec[l], jnp.float32)
                    for j in range(nseg):
                        rows[e, pl.ds(j * 16, 16)] = (
                            rows[e, pl.ds(j * 16, 16)] * ws)
                return c2

            lax.fori_loop(0, _CH // 16, scale, 0)
            pltpu.sync_copy(rows, acc.at[dstv], add=True)
            return carry

        lax.fori_loop(0, nch, chunk, 0)
        plsc.subcore_barrier()
        for j in range(2):
            c = sid + j * 16

            @pl.when(c < _NRCH)
            def _():
                pltpu.sync_copy(acc.at[pl.ds(c * _RCH, _RCH)],
                                out_hbm.at[pl.ds(cid * _N + c * _RCH, _RCH)])

    return k(*f_tables, src, dst, w)


def _mm_body(x_ref, w_ref, o_ref):
    o_ref[...] = jnp.dot(x_ref[...], w_ref[...],
                         preferred_element_type=jnp.float32)


def _tc_mm(x, w):
    m, kdim = x.shape
    n = w.shape[1]
    bm = 1000
    return pl.pallas_call(
        _mm_body,
        grid=(m // bm,),
        in_specs=[pl.BlockSpec((bm, kdim), lambda i: (i, 0)),
                  pl.BlockSpec((kdim, n), lambda i: (0, 0))],
        out_specs=pl.BlockSpec((bm, n), lambda i: (i, 0)),
        out_shape=jax.ShapeDtypeStruct((m, n), jnp.float32),
    )(x, w)


def _mid_body(a_ref, b_ref, m_ref, w_ref, o_ref):
    s = jnp.concatenate([a_ref[0], a_ref[1]], axis=-1) + b_ref[...]
    h = jnp.maximum(s, 0.0) * m_ref[...]
    o_ref[...] = jnp.dot(h, w_ref[...], preferred_element_type=jnp.float32)


def _mid(agg, b1, mask, w2p):
    bm = 1000
    return pl.pallas_call(
        _mid_body,
        grid=(_N // bm,),
        in_specs=[pl.BlockSpec((2, bm, _DH), lambda i: (0, i, 0)),
                  pl.BlockSpec((1, _D), lambda i: (0, 0)),
                  pl.BlockSpec((bm, 1), lambda i: (i, 0)),
                  pl.BlockSpec((_D, _DP), lambda i: (0, 0))],
        out_specs=pl.BlockSpec((bm, _DP), lambda i: (i, 0)),
        out_shape=jax.ShapeDtypeStruct((_N, _DP), jnp.float32),
    )(agg, b1, mask, w2p)


def _out_body(a_ref, b_ref, m_ref, o_ref):
    s = a_ref[0] + a_ref[1] + b_ref[...]
    o_ref[...] = jax.nn.sigmoid(s[:, :2]) * m_ref[...]


def _out(agg, b2p, mask):
    bm = 1000
    return pl.pallas_call(
        _out_body,
        grid=(_N // bm,),
        in_specs=[pl.BlockSpec((2, bm, _DP), lambda i: (0, i, 0)),
                  pl.BlockSpec((1, _DP), lambda i: (0, 0)),
                  pl.BlockSpec((bm, 1), lambda i: (i, 0))],
        out_specs=pl.BlockSpec((bm, 2), lambda i: (i, 0)),
        out_shape=jax.ShapeDtypeStruct((_N, 2), jnp.float32),
    )(agg, b2p, mask)


def kernel(x, edge_index, edge_weight, mask, W1, b1, W2, b2):
    src = edge_index[0].astype(jnp.int32)
    dst = edge_index[1].astype(jnp.int32)
    w = edge_weight.astype(jnp.float32)

    h = _tc_mm(x, W1)
    agg1 = _spmm(_DH, [h[:, :_DH], h[:, _DH:]], src, dst, w,
                 edge_split=False).reshape(2, _N, _DH)

    w2p = jnp.zeros((_D, _DP), jnp.float32).at[:, :2].set(W2)
    p = _mid(agg1, b1.reshape(1, _D), mask, w2p)

    agg2 = _spmm(_DP, [p], src, dst, w,
                 edge_split=True).reshape(2, _N, _DP)
    b2p = jnp.zeros((1, _DP), jnp.float32).at[0, :2].set(b2)
    return _out(agg2, b2p, mask)
